# Initial kernel scaffold; baseline (speedup 1.0000x reference)
import functools

import jax
import jax.numpy as jnp
from jax import lax
from jax.experimental import pallas as pl
from jax.experimental.pallas import tpu as pltpu
from jax.experimental.pallas import tpu_sc as plsc

_B, _S, _H = 1024, 200, 128
_N = _B * _S
_NC, _NS = 2, 16
_NW = _NC * _NS
_PER_W = _N // _NW
_CH = 128
_NCH = _PER_W // _CH
_K = 5
_NOUT = _NCH // _K

_mesh = plsc.VectorSubcoreMesh(core_axis_name="c", subcore_axis_name="s")


@functools.partial(
    pl.kernel,
    out_type=jax.ShapeDtypeStruct((_N, _H), jnp.float32),
    mesh=_mesh,
    scratch_types=[
        pltpu.VMEM((_NCH, _CH), jnp.int32),
        pltpu.VMEM((_K * _CH, _H), jnp.float32),
        pltpu.SemaphoreType.DMA,
    ],
)
def _embed_gather(idx_hbm, tbl_hbm, out_hbm, idx_v, rows_v, sem):
    wid = lax.axis_index("s") * _NC + lax.axis_index("c")
    blk0 = wid * _NCH
    pltpu.sync_copy(idx_hbm.at[pl.ds(blk0, _NCH)], idx_v)

    @pl.loop(0, _NOUT)
    def _outer(g):
        copies = []
        for b in range(_K):
            copies.append(
                pltpu.async_copy(
                    tbl_hbm.at[idx_v.at[g * _K + b]],
                    rows_v.at[pl.ds(b * _CH, _CH)],
                    sem,
                )
            )
        for c in copies:
            c.wait()
        pltpu.sync_copy(
            rows_v,
            out_hbm.at[pl.ds((blk0 + g * _K) * _CH, _K * _CH)],
        )


def kernel(input_ids, token_table):
    idx = input_ids.reshape(_N // _CH, _CH).astype(jnp.int32)
    out = _embed_gather(idx, token_table)
    return out.reshape(_B, _S, _H)


# speedup vs baseline: 7.7185x; 7.7185x over previous
import functools

import jax
import jax.numpy as jnp
from jax import lax
from jax.experimental import pallas as pl
from jax.experimental.pallas import tpu as pltpu
from jax.experimental.pallas import tpu_sc as plsc

_B, _S, _H = 1024, 200, 128
_N = _B * _S
_NC, _NS = 2, 16
_NW = _NC * _NS
_PER_W = _N // _NW
_CH = 128
_NCH = _PER_W // _CH
_K = 5
_NOUT = _NCH // _K

_mesh = plsc.VectorSubcoreMesh(core_axis_name="c", subcore_axis_name="s")


@functools.partial(
    pl.kernel,
    out_type=jax.ShapeDtypeStruct((_N, _H), jnp.float32),
    mesh=_mesh,
    scratch_types=[
        pltpu.VMEM((_NCH, _CH), jnp.int32),
        pltpu.VMEM((_K * _CH, _H), jnp.float32),
        pltpu.SemaphoreType.DMA,
    ],
)
def _embed_gather(idx_hbm, tbl_hbm, out_hbm, idx_v, rows_v, sem):
    wid = lax.axis_index("s") * _NC + lax.axis_index("c")
    blk0 = wid * _NCH
    pltpu.sync_copy(idx_hbm.at[wid], idx_v)

    @pl.loop(0, _NOUT)
    def _outer(g):
        copies = []
        for b in range(_K):
            copies.append(
                pltpu.async_copy(
                    tbl_hbm.at[idx_v.at[g * _K + b]],
                    rows_v.at[pl.ds(b * _CH, _CH)],
                    sem,
                )
            )
        for c in copies:
            c.wait()
        pltpu.sync_copy(
            rows_v,
            out_hbm.at[pl.ds((blk0 + g * _K) * _CH, _K * _CH)],
        )


def kernel(input_ids, token_table):
    idx = input_ids.reshape(_NW, _NCH, _CH).astype(jnp.int32)
    out = _embed_gather(idx, token_table)
    return out.reshape(_B, _S, _H)


# speedup vs baseline: 8.0190x; 1.0389x over previous
import functools

import jax
import jax.numpy as jnp
from jax import lax
from jax.experimental import pallas as pl
from jax.experimental.pallas import tpu as pltpu
from jax.experimental.pallas import tpu_sc as plsc

_B, _S, _H = 1024, 200, 128
_N = _B * _S
_NC, _NS = 2, 16
_NW = _NC * _NS
_PER_W = _N // _NW
_CH = 64
_NCH = _PER_W // _CH
_K = 5
_G = _K * _CH
_NG = _NCH // _K

_mesh = plsc.VectorSubcoreMesh(core_axis_name="c", subcore_axis_name="s")


@functools.partial(
    pl.kernel,
    out_type=jax.ShapeDtypeStruct((_N, _H), jnp.float32),
    mesh=_mesh,
    scratch_types=[
        pltpu.VMEM((_NCH, _CH), jnp.int32),
        pltpu.VMEM((2, _G, _H), jnp.float32),
        pltpu.SemaphoreType.DMA,
        pltpu.SemaphoreType.DMA,
    ],
)
def _embed_gather(idx_hbm, tbl_hbm, out_hbm, idx_v, rows_v, gsem, wsem):
    wid = lax.axis_index("s") * _NC + lax.axis_index("c")
    row0 = wid * _PER_W
    pltpu.sync_copy(idx_hbm.at[wid], idx_v)

    def fire(gi, p):
        for b in range(_K):
            pltpu.async_copy(
                tbl_hbm.at[idx_v.at[gi * _K + b]],
                rows_v.at[p].at[pl.ds(b * _CH, _CH)],
                gsem,
            )

    def drain_gathers(p):
        pltpu.make_async_copy(
            tbl_hbm.at[pl.ds(0, _G)], rows_v.at[p], gsem
        ).wait()

    def writeback(gi, p):
        return pltpu.async_copy(
            rows_v.at[p],
            out_hbm.at[pl.ds(row0 + gi * _G, _G)],
            wsem,
        )

    fire(0, 0)
    fire(1, 1)

    @pl.loop(0, _NG - 2, step=2)
    def _steady(g):
        for p in range(2):
            gi = g + p
            drain_gathers(p)
            wb = writeback(gi, p)
            wb.wait()
            fire(gi + 2, p)

    for p in range(2):
        gi = _NG - 2 + p
        drain_gathers(p)
        writeback(gi, p).wait()


def kernel(input_ids, token_table):
    idx = input_ids.reshape(_NW, _NCH, _CH).astype(jnp.int32)
    out = _embed_gather(idx, token_table)
    return out.reshape(_B, _S, _H)


# speedup vs baseline: 8.1299x; 1.0138x over previous
import functools

import jax
import jax.numpy as jnp
from jax import lax
from jax.experimental import pallas as pl
from jax.experimental.pallas import tpu as pltpu
from jax.experimental.pallas import tpu_sc as plsc

_B, _S, _H = 1024, 200, 128
_N = _B * _S
_NC, _NS = 2, 16
_NW = _NC * _NS
_PER_W = _N // _NW
_CH = 128
_NCH = _PER_W // _CH
_NBUF = 4

_mesh = plsc.VectorSubcoreMesh(core_axis_name="c", subcore_axis_name="s")


@functools.partial(
    pl.kernel,
    out_type=jax.ShapeDtypeStruct((_N, _H), jnp.float32),
    mesh=_mesh,
    scratch_types=[
        pltpu.VMEM((_NCH, _CH), jnp.int32),
        pltpu.VMEM((_NBUF, _CH, _H), jnp.float32),
        pltpu.SemaphoreType.DMA,
        pltpu.SemaphoreType.DMA,
    ],
)
def _embed_gather(idx_hbm, tbl_hbm, out_hbm, idx_v, rows_v, gsem, wsem):
    wid = lax.axis_index("s") * _NC + lax.axis_index("c")
    row0 = wid * _PER_W
    pltpu.sync_copy(idx_hbm.at[wid], idx_v)

    def fire(s, p):
        pltpu.async_copy(tbl_hbm.at[idx_v.at[s]], rows_v.at[p], gsem)

    def drain_gather(p):
        pltpu.make_async_copy(
            tbl_hbm.at[pl.ds(0, _CH)], rows_v.at[p], gsem
        ).wait()

    def start_wb(s, p):
        pltpu.async_copy(
            rows_v.at[p], out_hbm.at[pl.ds(row0 + s * _CH, _CH)], wsem
        )

    def drain_wb():
        pltpu.make_async_copy(
            rows_v.at[0], out_hbm.at[pl.ds(0, _CH)], wsem
        ).wait()

    def stage(s, p, wbwait, nfire):
        drain_gather(p)
        if wbwait:
            drain_wb()
        start_wb(s, p)
        if nfire:
            fire(s + 2, (p + 2) % _NBUF)

    fire(0, 0)
    fire(1, 1)
    stage(0, 0, wbwait=False, nfire=True)
    stage(1, 1, wbwait=False, nfire=True)

    @pl.loop(2, _NCH - 4, step=_NBUF)
    def _steady(g):
        for i in range(_NBUF):
            stage(g + i, (2 + i) % _NBUF, wbwait=True, nfire=True)

    stage(_NCH - 4, (_NCH - 4) % _NBUF, wbwait=True, nfire=True)
    stage(_NCH - 3, (_NCH - 3) % _NBUF, wbwait=True, nfire=True)
    stage(_NCH - 2, (_NCH - 2) % _NBUF, wbwait=True, nfire=False)
    stage(_NCH - 1, (_NCH - 1) % _NBUF, wbwait=True, nfire=False)
    drain_wb()
    drain_wb()


def kernel(input_ids, token_table):
    idx = input_ids.reshape(_NW, _NCH, _CH).astype(jnp.int32)
    out = _embed_gather(idx, token_table)
    return out.reshape(_B, _S, _H)


# speedup vs baseline: 8.2059x; 1.0093x over previous
import functools

import jax
import jax.numpy as jnp
from jax import lax
from jax.experimental import pallas as pl
from jax.experimental.pallas import tpu as pltpu
from jax.experimental.pallas import tpu_sc as plsc

_B, _S, _H = 1024, 200, 128
_N = _B * _S
_NC, _NS = 2, 16
_NW = _NC * _NS
_PER_W = _N // _NW
_CH = 128
_NCH = _PER_W // _CH
_NBUF = 6
_GDEPTH = 4

_mesh = plsc.VectorSubcoreMesh(core_axis_name="c", subcore_axis_name="s")


@functools.partial(
    pl.kernel,
    out_type=jax.ShapeDtypeStruct((_N, _H), jnp.float32),
    mesh=_mesh,
    scratch_types=[
        pltpu.VMEM((_NCH, _CH), jnp.int32),
        pltpu.VMEM((_NBUF, _CH, _H), jnp.float32),
        pltpu.SemaphoreType.DMA,
        pltpu.SemaphoreType.DMA,
    ],
)
def _embed_gather(idx_hbm, tbl_hbm, out_hbm, idx_v, rows_v, gsem, wsem):
    wid = lax.axis_index("s") * _NC + lax.axis_index("c")
    row0 = wid * _PER_W
    pltpu.sync_copy(idx_hbm.at[wid], idx_v)

    def fire(s, p):
        pltpu.async_copy(tbl_hbm.at[idx_v.at[s]], rows_v.at[p], gsem)

    def drain_gather(p):
        pltpu.make_async_copy(
            tbl_hbm.at[pl.ds(0, _CH)], rows_v.at[p], gsem
        ).wait()

    def start_wb(s, p):
        pltpu.async_copy(
            rows_v.at[p], out_hbm.at[pl.ds(row0 + s * _CH, _CH)], wsem
        )

    def drain_wb():
        pltpu.make_async_copy(
            rows_v.at[0], out_hbm.at[pl.ds(0, _CH)], wsem
        ).wait()

    def stage(s, p, wbwait, nfire):
        drain_gather(p)
        if wbwait:
            drain_wb()
        start_wb(s, p)
        if nfire:
            fire(s + _GDEPTH, (p + _GDEPTH) % _NBUF)

    for s in range(_GDEPTH):
        fire(s, s)
    stage(0, 0, wbwait=False, nfire=True)
    stage(1, 1, wbwait=False, nfire=True)

    @pl.loop(2, 44, step=_NBUF)
    def _steady(g):
        for i in range(_NBUF):
            stage(g + i, (2 + i) % _NBUF, wbwait=True, nfire=True)

    stage(44, 44 % _NBUF, wbwait=True, nfire=True)
    stage(45, 45 % _NBUF, wbwait=True, nfire=True)
    for s in range(46, 50):
        stage(s, s % _NBUF, wbwait=True, nfire=False)
    drain_wb()
    drain_wb()


def kernel(input_ids, token_table):
    idx = input_ids.reshape(_NW, _NCH, _CH).astype(jnp.int32)
    out = _embed_gather(idx, token_table)
    return out.reshape(_B, _S, _H)
